# Initial kernel scaffold; baseline (speedup 1.0000x reference)
import jax
import jax.numpy as jnp
from jax.experimental import pallas as pl


def kernel(z, codebook):
    raise NotImplementedError("write your pallas kernel here")


# speedup vs baseline: 1.1966x; 1.1966x over previous
import functools

import jax
import jax.numpy as jnp
from jax import lax
from jax.experimental import pallas as pl
from jax.experimental.pallas import tpu as pltpu
from jax.experimental.pallas import tpu_sc as plsc

_TILE = 256


def _dist_argmin_kernel(z_ref, cb_ref, a2_ref, b2_ref, idx_ref, dsum_ref):
    i = pl.program_id(0)
    ab = lax.dot_general(
        z_ref[...], cb_ref[...], (((1,), (1,)), ((), ())),
        preferred_element_type=jnp.float32)
    d = a2_ref[...] - 2 * ab
    d = d + b2_ref[...]
    idx_ref[...] = jnp.argmin(d, axis=1).astype(jnp.int32)
    dmin = jnp.min(d, axis=1)

    @pl.when(i == 0)
    def _():
        dsum_ref[0, 0] = 0.0

    dsum_ref[0, 0] += jnp.sum(dmin)


def _distances_argmin(z_flat, codebook, a2, b2):
    n, d = z_flat.shape
    k = codebook.shape[0]
    grid = n // _TILE
    return pl.pallas_call(
        _dist_argmin_kernel,
        grid=(grid,),
        in_specs=[
            pl.BlockSpec((_TILE, d), lambda i: (i, 0)),
            pl.BlockSpec((k, d), lambda i: (0, 0)),
            pl.BlockSpec((_TILE, 1), lambda i: (i, 0)),
            pl.BlockSpec((1, k), lambda i: (0, 0)),
        ],
        out_specs=[
            pl.BlockSpec((_TILE,), lambda i: (i,)),
            pl.BlockSpec(memory_space=pltpu.SMEM, block_shape=(1, 1),
                         index_map=lambda i: (0, 0)),
        ],
        out_shape=[
            jax.ShapeDtypeStruct((n,), jnp.int32),
            jax.ShapeDtypeStruct((1, 1), jnp.float32),
        ],
    )(z_flat, codebook, a2, b2)


def _sc_gather(table, idx):
    v, d = table.shape
    b = idx.shape[0]
    info = plsc.get_sparse_core_info()
    nw = info.num_cores * info.num_subcores
    b_per_w = b // nw
    mesh = plsc.VectorSubcoreMesh(core_axis_name="c", subcore_axis_name="s")

    @functools.partial(
        pl.kernel, mesh=mesh,
        compiler_params=pltpu.CompilerParams(use_tc_tiling_on_sc=False),
        out_type=jax.ShapeDtypeStruct((b, d), jnp.float32),
        scratch_types=[
            pltpu.VMEM((b_per_w,), jnp.int32),
            pltpu.VMEM((b_per_w, d), jnp.float32),
            pltpu.SemaphoreType.DMA,
        ],
    )
    def k(table_hbm, idx_hbm, out_hbm, idx_v, rows_v, sem):
        wid = lax.axis_index("s") * info.num_cores + lax.axis_index("c")
        base = wid * b_per_w
        pltpu.sync_copy(idx_hbm.at[pl.ds(base, b_per_w)], idx_v)
        pltpu.async_copy(table_hbm.at[idx_v], rows_v, sem).wait()
        pltpu.sync_copy(rows_v, out_hbm.at[pl.ds(base, b_per_w)])

    return k(table, idx)


def kernel(z, codebook):
    d = z.shape[-1]
    z_flat = z.reshape(-1, d)
    n = z_flat.shape[0]
    a2 = jnp.sum(z_flat ** 2, axis=1, keepdims=True)
    b2 = jnp.sum(codebook.T ** 2, axis=0, keepdims=True)
    idx, dsum = _distances_argmin(z_flat, codebook, a2, b2)
    quantized = _sc_gather(codebook, idx).reshape(z.shape)
    loss = dsum[0, 0] * (1.25 / (n * d))
    quantized_st = z + (quantized - z)
    return quantized_st, loss


# speedup vs baseline: 1.2999x; 1.0863x over previous
import functools

import jax
import jax.numpy as jnp
from jax import lax
from jax.experimental import pallas as pl
from jax.experimental.pallas import tpu as pltpu
from jax.experimental.pallas import tpu_sc as plsc

_TILE = 256


def _dist_argmin_kernel(z_ref, cbm2_ref, a2_ref, b2_ref, ii_ref, idx_ref,
                        dsum_ref):
    i = pl.program_id(0)
    k = cbm2_ref.shape[0]
    ab2 = lax.dot_general(
        z_ref[...], cbm2_ref[...], (((1,), (1,)), ((), ())),
        preferred_element_type=jnp.float32)
    d = (a2_ref[...] + ab2) + b2_ref[...]
    dmin = jnp.min(d, axis=1, keepdims=True)
    idx_f = jnp.min(jnp.where(d == dmin, ii_ref[...], float(k)), axis=1)
    idx_ref[...] = idx_f.astype(jnp.int32)

    @pl.when(i == 0)
    def _():
        dsum_ref[0, 0] = 0.0

    dsum_ref[0, 0] += jnp.sum(dmin)


def _distances_argmin(z_flat, codebook, a2, b2):
    n, d = z_flat.shape
    k = codebook.shape[0]
    grid = n // _TILE
    return pl.pallas_call(
        _dist_argmin_kernel,
        grid=(grid,),
        in_specs=[
            pl.BlockSpec((_TILE, d), lambda i: (i, 0)),
            pl.BlockSpec((k, d), lambda i: (0, 0)),
            pl.BlockSpec((_TILE, 1), lambda i: (i, 0)),
            pl.BlockSpec((1, k), lambda i: (0, 0)),
            pl.BlockSpec((1, k), lambda i: (0, 0)),
        ],
        out_specs=[
            pl.BlockSpec((_TILE,), lambda i: (i,)),
            pl.BlockSpec(memory_space=pltpu.SMEM, block_shape=(1, 1),
                         index_map=lambda i: (0, 0)),
        ],
        out_shape=[
            jax.ShapeDtypeStruct((n,), jnp.int32),
            jax.ShapeDtypeStruct((1, 1), jnp.float32),
        ],
    )(z_flat, codebook, a2, b2,
      jnp.arange(k, dtype=jnp.float32).reshape(1, k))


def _sc_gather(table, idx):
    v, d = table.shape
    b = idx.shape[0]
    info = plsc.get_sparse_core_info()
    nw = info.num_cores * info.num_subcores
    b_per_w = b // nw
    mesh = plsc.VectorSubcoreMesh(core_axis_name="c", subcore_axis_name="s")

    @functools.partial(
        pl.kernel, mesh=mesh,
        compiler_params=pltpu.CompilerParams(use_tc_tiling_on_sc=False),
        out_type=jax.ShapeDtypeStruct((b, d), jnp.float32),
        scratch_types=[
            pltpu.VMEM((b_per_w,), jnp.int32),
            pltpu.VMEM((b_per_w, d), jnp.float32),
            pltpu.SemaphoreType.DMA,
        ],
    )
    def k(table_hbm, idx_hbm, out_hbm, idx_v, rows_v, sem):
        wid = lax.axis_index("s") * info.num_cores + lax.axis_index("c")
        base = wid * b_per_w
        pltpu.sync_copy(idx_hbm.at[pl.ds(base, b_per_w)], idx_v)
        pltpu.async_copy(table_hbm.at[idx_v], rows_v, sem).wait()
        pltpu.sync_copy(rows_v, out_hbm.at[pl.ds(base, b_per_w)])

    return k(table, idx)


def kernel(z, codebook):
    d = z.shape[-1]
    z_flat = z.reshape(-1, d)
    n = z_flat.shape[0]
    a2 = jnp.sum(z_flat ** 2, axis=1, keepdims=True)
    b2 = jnp.sum(codebook.T ** 2, axis=0, keepdims=True)
    idx, dsum = _distances_argmin(z_flat, -2.0 * codebook, a2, b2)
    quantized = _sc_gather(codebook, idx).reshape(z.shape)
    loss = dsum[0, 0] * (1.25 / (n * d))
    quantized_st = z + (quantized - z)
    return quantized_st, loss


# speedup vs baseline: 1.4504x; 1.1158x over previous
import functools

import jax
import jax.numpy as jnp
from jax import lax
from jax.experimental import pallas as pl
from jax.experimental.pallas import tpu as pltpu
from jax.experimental.pallas import tpu_sc as plsc

_TILE = 512


def _dist_argmin_kernel(z_ref, cbm2_ref, a2_ref, b2_ref, ii_ref, idx_ref,
                        dsum_ref):
    i = pl.program_id(0)
    k = cbm2_ref.shape[0]
    ab2 = lax.dot_general(
        z_ref[...], cbm2_ref[...], (((1,), (1,)), ((), ())),
        preferred_element_type=jnp.float32)
    d = (a2_ref[...] + ab2) + b2_ref[...]
    dmin = jnp.min(d, axis=1, keepdims=True)
    idx_f = jnp.min(jnp.where(d == dmin, ii_ref[...], float(k)), axis=1)
    idx_ref[...] = idx_f.astype(jnp.int32)

    @pl.when(i == 0)
    def _():
        dsum_ref[0, 0] = 0.0

    dsum_ref[0, 0] += jnp.sum(dmin)


def _distances_argmin(z_flat, codebook, a2, b2):
    n, d = z_flat.shape
    k = codebook.shape[0]
    grid = n // _TILE
    return pl.pallas_call(
        _dist_argmin_kernel,
        grid=(grid,),
        in_specs=[
            pl.BlockSpec((_TILE, d), lambda i: (i, 0)),
            pl.BlockSpec((k, d), lambda i: (0, 0)),
            pl.BlockSpec((_TILE, 1), lambda i: (i, 0)),
            pl.BlockSpec((1, k), lambda i: (0, 0)),
            pl.BlockSpec((1, k), lambda i: (0, 0)),
        ],
        out_specs=[
            pl.BlockSpec((_TILE,), lambda i: (i,)),
            pl.BlockSpec(memory_space=pltpu.SMEM, block_shape=(1, 1),
                         index_map=lambda i: (0, 0)),
        ],
        out_shape=[
            jax.ShapeDtypeStruct((n,), jnp.int32),
            jax.ShapeDtypeStruct((1, 1), jnp.float32),
        ],
    )(z_flat, codebook, a2, b2,
      jnp.arange(k, dtype=jnp.float32).reshape(1, k))


def _sc_gather(table, idx):
    v, d = table.shape
    b = idx.shape[0]
    info = plsc.get_sparse_core_info()
    nw = info.num_cores * info.num_subcores
    b_per_w = b // nw
    mesh = plsc.VectorSubcoreMesh(core_axis_name="c", subcore_axis_name="s")

    @functools.partial(
        pl.kernel, mesh=mesh,
        compiler_params=pltpu.CompilerParams(use_tc_tiling_on_sc=False),
        out_type=jax.ShapeDtypeStruct((b, d), jnp.float32),
        scratch_types=[
            pltpu.VMEM((b_per_w,), jnp.int32),
            pltpu.VMEM((b_per_w, d), jnp.float32),
            pltpu.SemaphoreType.DMA,
        ],
    )
    def k(table_hbm, idx_hbm, out_hbm, idx_v, rows_v, sem):
        wid = lax.axis_index("s") * info.num_cores + lax.axis_index("c")
        base = wid * b_per_w
        pltpu.sync_copy(idx_hbm.at[pl.ds(base, b_per_w)], idx_v)
        pltpu.async_copy(table_hbm.at[idx_v], rows_v, sem).wait()
        pltpu.sync_copy(rows_v, out_hbm.at[pl.ds(base, b_per_w)])

    return k(table, idx)


def kernel(z, codebook):
    d = z.shape[-1]
    z_flat = z.reshape(-1, d)
    n = z_flat.shape[0]
    a2 = jnp.sum(z_flat ** 2, axis=1, keepdims=True)
    b2 = jnp.sum(codebook.T ** 2, axis=0, keepdims=True)
    idx, dsum = _distances_argmin(z_flat, -2.0 * codebook, a2, b2)
    quantized = _sc_gather(codebook, idx).reshape(z.shape)
    loss = dsum[0, 0] * (1.25 / (n * d))
    return quantized, loss


# speedup vs baseline: 1.4831x; 1.0225x over previous
import functools

import jax
import jax.numpy as jnp
from jax import lax
from jax.experimental import pallas as pl
from jax.experimental.pallas import tpu as pltpu
from jax.experimental.pallas import tpu_sc as plsc

_TILE = 1024


def _dist_argmin_kernel(z_ref, cbm2_ref, a2_ref, b2_ref, ii_ref, idx_ref,
                        dsum_ref):
    i = pl.program_id(0)
    k = cbm2_ref.shape[0]
    ab2 = lax.dot_general(
        z_ref[...], cbm2_ref[...], (((1,), (1,)), ((), ())),
        preferred_element_type=jnp.float32)
    d = (a2_ref[...] + ab2) + b2_ref[...]
    dmin = jnp.min(d, axis=1, keepdims=True)
    idx_f = jnp.min(jnp.where(d == dmin, ii_ref[...], float(k)), axis=1)
    idx_ref[...] = idx_f.astype(jnp.int32)

    @pl.when(i == 0)
    def _():
        dsum_ref[0, 0] = 0.0

    dsum_ref[0, 0] += jnp.sum(dmin)


def _distances_argmin(z_flat, codebook, a2, b2):
    n, d = z_flat.shape
    k = codebook.shape[0]
    grid = n // _TILE
    return pl.pallas_call(
        _dist_argmin_kernel,
        grid=(grid,),
        compiler_params=pltpu.CompilerParams(
            vmem_limit_bytes=100 * 1024 * 1024),
        in_specs=[
            pl.BlockSpec((_TILE, d), lambda i: (i, 0)),
            pl.BlockSpec((k, d), lambda i: (0, 0)),
            pl.BlockSpec((_TILE, 1), lambda i: (i, 0)),
            pl.BlockSpec((1, k), lambda i: (0, 0)),
            pl.BlockSpec((1, k), lambda i: (0, 0)),
        ],
        out_specs=[
            pl.BlockSpec((_TILE,), lambda i: (i,)),
            pl.BlockSpec(memory_space=pltpu.SMEM, block_shape=(1, 1),
                         index_map=lambda i: (0, 0)),
        ],
        out_shape=[
            jax.ShapeDtypeStruct((n,), jnp.int32),
            jax.ShapeDtypeStruct((1, 1), jnp.float32),
        ],
    )(z_flat, codebook, a2, b2,
      jnp.arange(k, dtype=jnp.float32).reshape(1, k))


def _sc_gather(table, idx):
    v, d = table.shape
    b = idx.shape[0]
    info = plsc.get_sparse_core_info()
    nw = info.num_cores * info.num_subcores
    b_per_w = b // nw
    mesh = plsc.VectorSubcoreMesh(core_axis_name="c", subcore_axis_name="s")

    @functools.partial(
        pl.kernel, mesh=mesh,
        compiler_params=pltpu.CompilerParams(use_tc_tiling_on_sc=False),
        out_type=jax.ShapeDtypeStruct((b, d), jnp.float32),
        scratch_types=[
            pltpu.VMEM((b_per_w,), jnp.int32),
            pltpu.VMEM((b_per_w, d), jnp.float32),
            pltpu.SemaphoreType.DMA,
        ],
    )
    def k(table_hbm, idx_hbm, out_hbm, idx_v, rows_v, sem):
        wid = lax.axis_index("s") * info.num_cores + lax.axis_index("c")
        base = wid * b_per_w
        pltpu.sync_copy(idx_hbm.at[pl.ds(base, b_per_w)], idx_v)
        pltpu.async_copy(table_hbm.at[idx_v], rows_v, sem).wait()
        pltpu.sync_copy(rows_v, out_hbm.at[pl.ds(base, b_per_w)])

    return k(table, idx)


def kernel(z, codebook):
    d = z.shape[-1]
    z_flat = z.reshape(-1, d)
    n = z_flat.shape[0]
    a2 = jnp.sum(z_flat ** 2, axis=1, keepdims=True)
    b2 = jnp.sum(codebook.T ** 2, axis=0, keepdims=True)
    idx, dsum = _distances_argmin(z_flat, -2.0 * codebook, a2, b2)
    quantized = _sc_gather(codebook, idx).reshape(z.shape)
    loss = dsum[0, 0] * (1.25 / (n * d))
    return quantized, loss


# speedup vs baseline: 1.5548x; 1.0484x over previous
import functools

import jax
import jax.numpy as jnp
from jax import lax
from jax.experimental import pallas as pl
from jax.experimental.pallas import tpu as pltpu
from jax.experimental.pallas import tpu_sc as plsc

_TILE = 512
_RSUB = 64
_LB = 128


def _dist_argmin_kernel(z_ref, cb_ref, a2_ref, b2_ref, idx_ref, dsum_ref,
                        ab2_ref):
    i = pl.program_id(0)
    k = cb_ref.shape[0]
    ab2_ref[...] = lax.dot_general(
        -2.0 * z_ref[...], cb_ref[...], (((1,), (1,)), ((), ())),
        preferred_element_type=jnp.float32)

    @pl.when(i == 0)
    def _():
        dsum_ref[0, 0] = 0.0

    nb = k // _LB
    lane = lax.broadcasted_iota(jnp.int32, (1, _LB), 1).astype(jnp.float32)
    total = jnp.float32(0.0)
    for r in range(_TILE // _RSUB):
        rows = pl.ds(r * _RSUB, _RSUB)
        a2s = a2_ref[rows, :]
        m = jnp.full((_RSUB, _LB), jnp.inf, jnp.float32)
        g = jnp.zeros((_RSUB, _LB), jnp.float32)
        for b in range(nb):
            db = (a2s + ab2_ref[rows, pl.ds(b * _LB, _LB)]) \
                 + b2_ref[:, pl.ds(b * _LB, _LB)]
            cond = db < m
            m = jnp.minimum(m, db)
            g = jnp.where(cond, jnp.float32(b), g)
        dmin = jnp.min(m, axis=1, keepdims=True)
        cand = jnp.where(m == dmin, g * float(_LB) + lane, float(k))
        idx_ref[pl.ds(r * _RSUB, _RSUB)] = jnp.min(cand, axis=1).astype(jnp.int32)
        total += jnp.sum(dmin)
    dsum_ref[0, 0] += total


def _distances_argmin(z_flat, codebook, a2, b2):
    n, d = z_flat.shape
    k = codebook.shape[0]
    grid = n // _TILE
    return pl.pallas_call(
        _dist_argmin_kernel,
        grid=(grid,),
        compiler_params=pltpu.CompilerParams(
            vmem_limit_bytes=100 * 1024 * 1024),
        in_specs=[
            pl.BlockSpec((_TILE, d), lambda i: (i, 0)),
            pl.BlockSpec((k, d), lambda i: (0, 0)),
            pl.BlockSpec((_TILE, 1), lambda i: (i, 0)),
            pl.BlockSpec((1, k), lambda i: (0, 0)),
        ],
        out_specs=[
            pl.BlockSpec((_TILE,), lambda i: (i,)),
            pl.BlockSpec(memory_space=pltpu.SMEM, block_shape=(1, 1),
                         index_map=lambda i: (0, 0)),
        ],
        out_shape=[
            jax.ShapeDtypeStruct((n,), jnp.int32),
            jax.ShapeDtypeStruct((1, 1), jnp.float32),
        ],
        scratch_shapes=[pltpu.VMEM((_TILE, k), jnp.float32)],
    )(z_flat, codebook, a2, b2)


def _sc_gather(table, idx):
    v, d = table.shape
    b = idx.shape[0]
    info = plsc.get_sparse_core_info()
    nw = info.num_cores * info.num_subcores
    b_per_w = b // nw
    mesh = plsc.VectorSubcoreMesh(core_axis_name="c", subcore_axis_name="s")

    @functools.partial(
        pl.kernel, mesh=mesh,
        compiler_params=pltpu.CompilerParams(use_tc_tiling_on_sc=False),
        out_type=jax.ShapeDtypeStruct((b, d), jnp.float32),
        scratch_types=[
            pltpu.VMEM((b_per_w,), jnp.int32),
            pltpu.VMEM((b_per_w, d), jnp.float32),
            pltpu.SemaphoreType.DMA,
        ],
    )
    def k(table_hbm, idx_hbm, out_hbm, idx_v, rows_v, sem):
        wid = lax.axis_index("s") * info.num_cores + lax.axis_index("c")
        base = wid * b_per_w
        pltpu.sync_copy(idx_hbm.at[pl.ds(base, b_per_w)], idx_v)
        pltpu.async_copy(table_hbm.at[idx_v], rows_v, sem).wait()
        pltpu.sync_copy(rows_v, out_hbm.at[pl.ds(base, b_per_w)])

    return k(table, idx)


def kernel(z, codebook):
    d = z.shape[-1]
    z_flat = z.reshape(-1, d)
    n = z_flat.shape[0]
    a2 = jnp.sum(z_flat ** 2, axis=1, keepdims=True)
    b2 = jnp.sum(codebook.T ** 2, axis=0, keepdims=True)
    idx, dsum = _distances_argmin(z_flat, codebook, a2, b2)
    quantized = _sc_gather(codebook, idx).reshape(z.shape)
    loss = dsum[0, 0] * (1.25 / (n * d))
    return quantized, loss


# speedup vs baseline: 1.7001x; 1.0934x over previous
import functools

import jax
import jax.numpy as jnp
from jax import lax
from jax.experimental import pallas as pl
from jax.experimental.pallas import tpu as pltpu
from jax.experimental.pallas import tpu_sc as plsc

_TILE = 512
_RSUB = 64
_LB = 128


def _dist_argmin_kernel(z_ref, cb_ref, b2_ref, idx_ref, dsum_ref, ab2_ref):
    i = pl.program_id(0)
    k = cb_ref.shape[0]
    zt = z_ref[...]
    a2_tile = jnp.sum(zt * zt, axis=1, keepdims=True)
    ab2_ref[...] = lax.dot_general(
        -2.0 * zt, cb_ref[...], (((1,), (1,)), ((), ())),
        preferred_element_type=jnp.float32)

    @pl.when(i == 0)
    def _():
        dsum_ref[0, 0] = 0.0

    nb = k // _LB
    lane = lax.broadcasted_iota(jnp.int32, (1, _LB), 1).astype(jnp.float32)
    total = jnp.float32(0.0)
    for r in range(_TILE // _RSUB):
        rows = pl.ds(r * _RSUB, _RSUB)
        a2s = a2_tile[r * _RSUB:(r + 1) * _RSUB, :]
        m = jnp.full((_RSUB, _LB), jnp.inf, jnp.float32)
        g = jnp.zeros((_RSUB, _LB), jnp.float32)
        for b in range(nb):
            db = (a2s + ab2_ref[rows, pl.ds(b * _LB, _LB)]) \
                 + b2_ref[:, pl.ds(b * _LB, _LB)]
            cond = db < m
            m = jnp.minimum(m, db)
            g = jnp.where(cond, jnp.float32(b), g)
        dmin = jnp.min(m, axis=1, keepdims=True)
        cand = jnp.where(m == dmin, g * float(_LB) + lane, float(k))
        idx_ref[pl.ds(r * _RSUB, _RSUB)] = jnp.min(cand, axis=1).astype(jnp.int32)
        total += jnp.sum(dmin)
    dsum_ref[0, 0] += total


def _distances_argmin(z_flat, codebook, b2):
    n, d = z_flat.shape
    k = codebook.shape[0]
    grid = n // _TILE
    return pl.pallas_call(
        _dist_argmin_kernel,
        grid=(grid,),
        compiler_params=pltpu.CompilerParams(
            vmem_limit_bytes=100 * 1024 * 1024),
        in_specs=[
            pl.BlockSpec((_TILE, d), lambda i: (i, 0)),
            pl.BlockSpec((k, d), lambda i: (0, 0)),
            pl.BlockSpec((1, k), lambda i: (0, 0)),
        ],
        out_specs=[
            pl.BlockSpec((_TILE,), lambda i: (i,)),
            pl.BlockSpec(memory_space=pltpu.SMEM, block_shape=(1, 1),
                         index_map=lambda i: (0, 0)),
        ],
        out_shape=[
            jax.ShapeDtypeStruct((n,), jnp.int32),
            jax.ShapeDtypeStruct((1, 1), jnp.float32),
        ],
        scratch_shapes=[pltpu.VMEM((_TILE, k), jnp.float32)],
    )(z_flat, codebook, b2)


def _sc_gather(table, idx):
    v, d = table.shape
    b = idx.shape[0]
    info = plsc.get_sparse_core_info()
    nw = info.num_cores * info.num_subcores
    b_per_w = b // nw
    mesh = plsc.VectorSubcoreMesh(core_axis_name="c", subcore_axis_name="s")

    @functools.partial(
        pl.kernel, mesh=mesh,
        compiler_params=pltpu.CompilerParams(use_tc_tiling_on_sc=False),
        out_type=jax.ShapeDtypeStruct((b, d), jnp.float32),
        scratch_types=[
            pltpu.VMEM((b_per_w,), jnp.int32),
            pltpu.VMEM((b_per_w, d), jnp.float32),
            pltpu.SemaphoreType.DMA,
        ],
    )
    def k(table_hbm, idx_hbm, out_hbm, idx_v, rows_v, sem):
        wid = lax.axis_index("s") * info.num_cores + lax.axis_index("c")
        base = wid * b_per_w
        pltpu.sync_copy(idx_hbm.at[pl.ds(base, b_per_w)], idx_v)
        pltpu.async_copy(table_hbm.at[idx_v], rows_v, sem).wait()
        pltpu.sync_copy(rows_v, out_hbm.at[pl.ds(base, b_per_w)])

    return k(table, idx)


def kernel(z, codebook):
    d = z.shape[-1]
    z_flat = z.reshape(-1, d)
    n = z_flat.shape[0]
    b2 = jnp.sum(codebook.T ** 2, axis=0, keepdims=True)
    idx, dsum = _distances_argmin(z_flat, codebook, b2)
    quantized = _sc_gather(codebook, idx).reshape(z.shape)
    loss = dsum[0, 0] * (1.25 / (n * d))
    return quantized, loss


# speedup vs baseline: 1.7253x; 1.0148x over previous
import functools

import jax
import jax.numpy as jnp
from jax import lax
from jax.experimental import pallas as pl
from jax.experimental.pallas import tpu as pltpu
from jax.experimental.pallas import tpu_sc as plsc

_TILE = 1024
_RSUB = 64
_LB = 128


def _dist_argmin_kernel(z_ref, cb_ref, b2_ref, idx_ref, dsum_ref, ab2_ref):
    i = pl.program_id(0)
    k = cb_ref.shape[0]
    zt = z_ref[...]
    a2_tile = jnp.sum(zt * zt, axis=1, keepdims=True)
    ab2_ref[...] = lax.dot_general(
        -2.0 * zt, cb_ref[...], (((1,), (1,)), ((), ())),
        preferred_element_type=jnp.float32)

    @pl.when(i == 0)
    def _():
        dsum_ref[0, 0] = 0.0

    nb = k // _LB
    lane = lax.broadcasted_iota(jnp.int32, (1, _LB), 1).astype(jnp.float32)
    total = jnp.float32(0.0)
    for r in range(_TILE // _RSUB):
        rows = pl.ds(r * _RSUB, _RSUB)
        a2s = a2_tile[r * _RSUB:(r + 1) * _RSUB, :]
        m = jnp.full((_RSUB, _LB), jnp.inf, jnp.float32)
        g = jnp.zeros((_RSUB, _LB), jnp.float32)
        for b in range(nb):
            db = (a2s + ab2_ref[rows, pl.ds(b * _LB, _LB)]) \
                 + b2_ref[:, pl.ds(b * _LB, _LB)]
            cond = db < m
            m = jnp.minimum(m, db)
            g = jnp.where(cond, jnp.float32(b), g)
        dmin = jnp.min(m, axis=1, keepdims=True)
        cand = jnp.where(m == dmin, g * float(_LB) + lane, float(k))
        idx_ref[pl.ds(r * _RSUB, _RSUB)] = jnp.min(cand, axis=1).astype(jnp.int32)
        total += jnp.sum(dmin)
    dsum_ref[0, 0] += total


def _distances_argmin(z_flat, codebook, b2):
    n, d = z_flat.shape
    k = codebook.shape[0]
    grid = n // _TILE
    return pl.pallas_call(
        _dist_argmin_kernel,
        grid=(grid,),
        compiler_params=pltpu.CompilerParams(
            vmem_limit_bytes=100 * 1024 * 1024),
        in_specs=[
            pl.BlockSpec((_TILE, d), lambda i: (i, 0)),
            pl.BlockSpec((k, d), lambda i: (0, 0)),
            pl.BlockSpec((1, k), lambda i: (0, 0)),
        ],
        out_specs=[
            pl.BlockSpec((_TILE,), lambda i: (i,)),
            pl.BlockSpec(memory_space=pltpu.SMEM, block_shape=(1, 1),
                         index_map=lambda i: (0, 0)),
        ],
        out_shape=[
            jax.ShapeDtypeStruct((n,), jnp.int32),
            jax.ShapeDtypeStruct((1, 1), jnp.float32),
        ],
        scratch_shapes=[pltpu.VMEM((_TILE, k), jnp.float32)],
    )(z_flat, codebook, b2)


def _sc_gather(table, idx):
    v, d = table.shape
    b = idx.shape[0]
    info = plsc.get_sparse_core_info()
    nw = info.num_cores * info.num_subcores
    b_per_w = b // nw
    mesh = plsc.VectorSubcoreMesh(core_axis_name="c", subcore_axis_name="s")

    @functools.partial(
        pl.kernel, mesh=mesh,
        compiler_params=pltpu.CompilerParams(use_tc_tiling_on_sc=False),
        out_type=jax.ShapeDtypeStruct((b, d), jnp.float32),
        scratch_types=[
            pltpu.VMEM((b_per_w,), jnp.int32),
            pltpu.VMEM((b_per_w, d), jnp.float32),
            pltpu.SemaphoreType.DMA,
        ],
    )
    def k(table_hbm, idx_hbm, out_hbm, idx_v, rows_v, sem):
        wid = lax.axis_index("s") * info.num_cores + lax.axis_index("c")
        base = wid * b_per_w
        pltpu.sync_copy(idx_hbm.at[pl.ds(base, b_per_w)], idx_v)
        pltpu.async_copy(table_hbm.at[idx_v], rows_v, sem).wait()
        pltpu.sync_copy(rows_v, out_hbm.at[pl.ds(base, b_per_w)])

    return k(table, idx)


def kernel(z, codebook):
    d = z.shape[-1]
    z_flat = z.reshape(-1, d)
    n = z_flat.shape[0]
    b2 = jnp.sum(codebook.T ** 2, axis=0, keepdims=True)
    idx, dsum = _distances_argmin(z_flat, codebook, b2)
    quantized = _sc_gather(codebook, idx).reshape(z.shape)
    loss = dsum[0, 0] * (1.25 / (n * d))
    return quantized, loss
